# baseline (speedup 1.0000x reference)
import jax
import jax.numpy as jnp
from jax.experimental import pallas as pl
from jax.experimental.pallas import tpu as pltpu


def _body(xt_ref, et_ref, o_ref, e_ref, prev_ref):
    i = pl.program_id(0)

    @pl.when(i == 0)
    def _():
        e_ref[...] = et_ref[...].T

    x = xt_ref[...].astype(jnp.float32)
    prod = jax.lax.dot_general(
        x, e_ref[...], (((0,), (0,)), ((), ())),
        preferred_element_type=jnp.float32,
    )
    carry = prev_ref[...]
    shifted = jnp.concatenate([prod[1:, :], carry], axis=0)
    o_ref[...] = shifted.T
    prev_ref[...] = prod[0:1, :]


def kernel(inputs, embeddings):
    M, K = inputs.shape
    _, N = embeddings.shape
    xt = inputs.T
    et = embeddings.T
    BN = 1024
    nblk = M // BN
    out_t = pl.pallas_call(
        _body,
        grid=(nblk,),
        in_specs=[
            pl.BlockSpec((K, BN), lambda i, n=nblk: (0, n - 1 - i)),
            pl.BlockSpec((N, K), lambda i: (0, 0)),
        ],
        out_specs=pl.BlockSpec((N, BN), lambda i, n=nblk: (0, n - 1 - i)),
        out_shape=jax.ShapeDtypeStruct((N, M - 1), jnp.float32),
        scratch_shapes=[
            pltpu.VMEM((K, N), jnp.float32),
            pltpu.VMEM((1, N), jnp.float32),
        ],
    )(xt, et)
    return out_t.T


# speedup vs baseline: 1.3483x; 1.3483x over previous
import jax
import jax.numpy as jnp
from jax.experimental import pallas as pl
from jax.experimental.pallas import tpu as pltpu


def _body(xt_ref, et_ref, o_ref, e_ref, prev_ref):
    i = pl.program_id(0)

    @pl.when(i == 0)
    def _():
        e_ref[...] = et_ref[...].T

    x = xt_ref[...].astype(jnp.float32)
    prod_t = jax.lax.dot_general(
        e_ref[...], x, (((0,), (0,)), ((), ())),
        preferred_element_type=jnp.float32,
    )
    carry = prev_ref[...]
    o_ref[...] = jnp.concatenate([prod_t[:, 1:], carry], axis=1)
    prev_ref[...] = prod_t[:, 0:1]


def kernel(inputs, embeddings):
    M, K = inputs.shape
    _, N = embeddings.shape
    xt = inputs.T
    et = embeddings.T
    BN = 2048
    nblk = M // BN
    out_t = pl.pallas_call(
        _body,
        grid=(nblk,),
        in_specs=[
            pl.BlockSpec((K, BN), lambda i, n=nblk: (0, n - 1 - i)),
            pl.BlockSpec((N, K), lambda i: (0, 0)),
        ],
        out_specs=pl.BlockSpec((N, BN), lambda i, n=nblk: (0, n - 1 - i)),
        out_shape=jax.ShapeDtypeStruct((N, M - 1), jnp.float32),
        scratch_shapes=[
            pltpu.VMEM((K, N), jnp.float32),
            pltpu.VMEM((N, 1), jnp.float32),
        ],
    )(xt, et)
    return out_t.T
